# baseline (speedup 1.0000x reference)
import functools

import jax
import jax.numpy as jnp
from jax import lax
from jax.experimental import pallas as pl
from jax.experimental.pallas import tpu as pltpu
from jax.experimental.pallas import tpu_sc as plsc

N_Z = 16
BATCH = 16384
N_GROUP = 1000000
NC = 2
NS = 16
NW = NC * NS
B_PER_W = BATCH // NW
CH = 16
N_CH = B_PER_W // CH


def _sc_body(x_hbm, idx_hbm, emb1_hbm, emb2_hbm, out_hbm,
             idx_v, a_v, b_v, x_v, o_v, sem_a, sem_b, sem_x):
    wid = lax.axis_index("s") * NC + lax.axis_index("c")
    base = wid * B_PER_W
    base_blk = base // 8

    pltpu.sync_copy(idx_hbm.at[pl.ds(base, B_PER_W)], idx_v)

    lanes = lax.iota(jnp.int32, N_Z)
    zeros = jnp.zeros((N_Z,), jnp.int32)
    xj = lanes // 8
    xs = lanes % 8

    def chunk_body(c, _):
        r0 = c * CH
        cp_x = pltpu.async_copy(
            x_hbm.at[pl.ds(base_blk + 2 * c, 2)], x_v, sem_x)
        idx16 = idx_v[pl.ds(r0, CH)]
        blk16 = lax.shift_right_logical(idx16, 3)
        cps = []
        for r in range(CH):
            blk = blk16[r]
            cps.append(pltpu.async_copy(
                emb1_hbm.at[blk], a_v.at[r], sem_a))
            cps.append(pltpu.async_copy(
                emb2_hbm.at[blk], b_v.at[r], sem_b))
        cp_x.wait()
        for cp in cps:
            cp.wait()
        sub16 = lax.bitwise_and(idx16, 7)
        acc = plsc.load_gather(b_v, [lanes, sub16, zeros])
        for col in range(N_Z):
            colv = jnp.full((N_Z,), col, jnp.int32)
            xc = plsc.load_gather(x_v, [xj, xs, colv])
            ac = plsc.load_gather(a_v, [lanes, sub16, colv])
            acc = acc + xc * ac
        o_v[pl.ds(r0, CH)] = acc
        return 0

    lax.fori_loop(0, N_CH, chunk_body, 0)
    pltpu.sync_copy(o_v, out_hbm.at[pl.ds(base, B_PER_W)])


@jax.jit
def _rand_effect(x3, idx, emb1_3, emb2_3):
    mesh = plsc.VectorSubcoreMesh(core_axis_name="c", subcore_axis_name="s")
    k = functools.partial(
        pl.kernel,
        out_type=jax.ShapeDtypeStruct((BATCH,), jnp.float32),
        mesh=mesh,
        compiler_params=pltpu.CompilerParams(needs_layout_passes=False),
        scratch_types=[
            pltpu.VMEM((B_PER_W,), jnp.int32),
            pltpu.VMEM((CH, 8, N_Z), jnp.float32),
            pltpu.VMEM((CH, 8, 1), jnp.float32),
            pltpu.VMEM((2, 8, N_Z), jnp.float32),
            pltpu.VMEM((B_PER_W,), jnp.float32),
            pltpu.SemaphoreType.DMA,
            pltpu.SemaphoreType.DMA,
            pltpu.SemaphoreType.DMA,
        ],
    )(_sc_body)
    return k(x3, idx, emb1_3, emb2_3)


def kernel(x, idx, emb1, emb2):
    x3 = x.reshape(BATCH // 8, 8, N_Z)
    emb1_3 = emb1.reshape(N_GROUP // 8, 8, N_Z)
    emb2_3 = emb2.reshape(N_GROUP // 8, 8, 1)
    out = _rand_effect(x3, idx.astype(jnp.int32), emb1_3, emb2_3)
    return out.reshape(BATCH, 1)
